# baseline (speedup 1.0000x reference)
import functools

import jax
import jax.numpy as jnp
from jax import lax
from jax.experimental import pallas as pl
from jax.experimental.pallas import tpu as pltpu
from jax.experimental.pallas import tpu_sc as plsc

N = 10000
E = 160000
D_IN = 256
HID = 512
D_OUT = 256

NC = 2
NS = 16
L = 16

NPAD = 10240
BN2 = NPAD // 8

K = 128
NCHUNK = 80
NBUF = 2
P = 2
CPP = NCHUNK // P
EPT = NCHUNK * K
EPAD = NS * EPT
JUNK = NPAD - 1

EPH = E // (NC * NS)


def _mesh():
    return plsc.VectorSubcoreMesh(
        core_axis_name="c", subcore_axis_name="s", num_cores=NC, num_subcores=NS
    )


@functools.partial(
    pl.kernel,
    out_type=jax.ShapeDtypeStruct((2, NS, NPAD), jnp.int32),
    mesh=_mesh(),
    scratch_types=[
        pltpu.VMEM((E // NS,), jnp.int32),
        pltpu.VMEM((NPAD,), jnp.int32),
    ],
    compiler_params=pltpu.CompilerParams(needs_layout_passes=False),
)
def _deg_kernel(adj_hbm, parts_hbm, ev, hist):
    c = lax.axis_index("c")
    s = lax.axis_index("s")
    pltpu.sync_copy(adj_hbm.at[c, s], ev)

    zeros = jnp.zeros((L,), jnp.int32)

    def zero_body(i, _):
        hist[pl.ds(pl.multiple_of(i * L, L), L)] = zeros
        return 0

    lax.fori_loop(0, NPAD // L, zero_body, 0)

    ones = jnp.ones((L,), jnp.int32)

    def scat_body(i, _):
        idx = ev[pl.ds(pl.multiple_of(i * L, L), L)]
        plsc.addupdate_scatter(hist, [idx], ones)
        return 0

    lax.fori_loop(0, (E // NS) // L, scat_body, 0)
    pltpu.sync_copy(hist, parts_hbm.at[c, s])


def _mm_body(x_ref, w1_ref, wg_ref, parts_ref, h2_ref, binv_ref):
    deg_s = jnp.sum(parts_ref[0].astype(jnp.float32), axis=0)
    a = lax.rsqrt(jnp.maximum(deg_s, 1.0))
    xs = x_ref[...] * a[:, None]
    h1 = jnp.maximum(jnp.dot(xs, w1_ref[...], preferred_element_type=jnp.float32), 0.0)
    hh = jnp.dot(h1, wg_ref[...], preferred_element_type=jnp.float32)
    h2_ref[0] = hh[:, :128]
    h2_ref[1] = hh[:, 128:]
    deg_d = jnp.sum(parts_ref[1].astype(jnp.float32), axis=0)
    binv_ref[...] = lax.rsqrt(jnp.maximum(deg_d, 1.0))[:, None]


def _matmul_stage(x_pad, W1, Wg, parts):
    return pl.pallas_call(
        _mm_body,
        grid=(NPAD // BN2,),
        in_specs=[
            pl.BlockSpec((BN2, D_IN), lambda i: (i, 0)),
            pl.BlockSpec((D_IN, HID), lambda i: (0, 0)),
            pl.BlockSpec((HID, D_OUT), lambda i: (0, 0)),
            pl.BlockSpec((2, NS, BN2), lambda i: (0, 0, i)),
        ],
        out_specs=[
            pl.BlockSpec((2, BN2, 128), lambda i: (0, i, 0)),
            pl.BlockSpec((BN2, 1), lambda i: (i, 0)),
        ],
        out_shape=[
            jax.ShapeDtypeStruct((2, NPAD, 128), jnp.float32),
            jax.ShapeDtypeStruct((NPAD, 1), jnp.float32),
        ],
    )(x_pad, W1, Wg, parts)


@functools.partial(
    pl.kernel,
    out_type=jax.ShapeDtypeStruct((2, NPAD, 128), jnp.float32),
    mesh=_mesh(),
    scratch_types=[
        pltpu.VMEM((NCHUNK, K), jnp.int32),
        pltpu.VMEM((NCHUNK, K), jnp.int32),
        pltpu.VMEM((NCHUNK, K), jnp.int32),
        pltpu.VMEM((K, 128), jnp.float32),
        pltpu.VMEM_SHARED((NPAD, 128), jnp.float32),
        pltpu.SemaphoreType.DMA,
    ],
    compiler_params=pltpu.CompilerParams(needs_layout_passes=False),
)
def _scatter_kernel(h_hbm, src_hbm, dst_hbm, zeros_hbm, acc_hbm,
                    srcv, dstv, gidx, rows, acc, sem):
    c = lax.axis_index("c")
    s = lax.axis_index("s")
    rpt = NPAD // NS

    pltpu.sync_copy(zeros_hbm, acc.at[pl.ds(s * rpt, rpt)])

    pltpu.sync_copy(src_hbm.at[s], srcv)
    pltpu.sync_copy(dst_hbm.at[s], dstv)

    off = c * NPAD

    def add_body(j, _):
        def inner(k, _):
            sl = pl.ds(pl.multiple_of(k * L, L), L)
            gidx[j, sl] = srcv[j, sl] + off
            return 0
        lax.fori_loop(0, K // L, inner, 0)
        return 0

    lax.fori_loop(0, NCHUNK, add_body, 0)

    plsc.subcore_barrier()

    def edge_body(j, _):
        pltpu.async_copy(h_hbm.at[gidx.at[j]], rows, sem).wait()
        pltpu.sync_copy(rows, acc.at[dstv.at[j]], add=True)
        return 0

    lax.fori_loop(0, NCHUNK, edge_body, 0)

    plsc.subcore_barrier()
    pltpu.sync_copy(acc.at[pl.ds(s * rpt, rpt)], acc_hbm.at[c, pl.ds(s * rpt, rpt)])


def _post_body(acc_ref, b_ref, out_ref):
    out_ref[...] = acc_ref[0].astype(jnp.float32) * b_ref[...]


def _post_stage(acc2, binv):
    return pl.pallas_call(
        _post_body,
        grid=(N // 1000, 2),
        in_specs=[
            pl.BlockSpec((1, 1000, 128), lambda i, j: (j, i, 0)),
            pl.BlockSpec((1000, 1), lambda i, j: (i, 0)),
        ],
        out_specs=pl.BlockSpec((1000, 128), lambda i, j: (i, j)),
        out_shape=jax.ShapeDtypeStruct((N, D_OUT), jnp.float32),
    )(acc2, binv)


def kernel(x, adj, W1, Wg):
    src = adj[0]
    dst = adj[1]
    adj16 = adj.reshape(2, NS, E // NS)
    src_r = jnp.concatenate(
        [src, jnp.zeros((EPAD - E,), jnp.int32)]).reshape(NS, NCHUNK, K)
    dst_r = jnp.concatenate(
        [dst, jnp.full((EPAD - E,), JUNK, jnp.int32)]).reshape(NS, NCHUNK, K)
    x_pad = jnp.pad(x, ((0, NPAD - N), (0, 0)))
    zeros_blk = jnp.zeros((NPAD // NS, 128), jnp.float32)

    parts = _deg_kernel(adj16)
    h2, binv = _matmul_stage(x_pad, W1, Wg, parts)
    h_flat = h2.reshape(2 * NPAD, 128)
    acc2 = _scatter_kernel(h_flat, src_r, dst_r, zeros_blk)
    out = _post_stage(acc2, binv)
    return out


# speedup vs baseline: 1.8342x; 1.8342x over previous
import functools

import jax
import jax.numpy as jnp
from jax import lax
from jax.experimental import pallas as pl
from jax.experimental.pallas import tpu as pltpu
from jax.experimental.pallas import tpu_sc as plsc

N = 10000
E = 160000
D_IN = 256
HID = 512
D_OUT = 256

NC = 2
NS = 16
L = 16

NPAD = 10240
BN2 = NPAD // 8

K = 128
NCHUNK = 80
NBUF = 2
P = 2
CPP = NCHUNK // P
EPT = NCHUNK * K
EPAD = NS * EPT
JUNK = NPAD - 1

EPH = E // (NC * NS)


def _mesh():
    return plsc.VectorSubcoreMesh(
        core_axis_name="c", subcore_axis_name="s", num_cores=NC, num_subcores=NS
    )


@functools.partial(
    pl.kernel,
    out_type=jax.ShapeDtypeStruct((2, NS, NPAD), jnp.int32),
    mesh=_mesh(),
    scratch_types=[
        pltpu.VMEM((E // NS,), jnp.int32),
        pltpu.VMEM((NPAD,), jnp.int32),
    ],
    compiler_params=pltpu.CompilerParams(needs_layout_passes=False),
)
def _deg_kernel(adj_hbm, parts_hbm, ev, hist):
    c = lax.axis_index("c")
    s = lax.axis_index("s")
    pltpu.sync_copy(adj_hbm.at[c, s], ev)

    zeros = jnp.zeros((L,), jnp.int32)

    def zero_body(i, _):
        hist[pl.ds(pl.multiple_of(i * L, L), L)] = zeros
        return 0

    lax.fori_loop(0, NPAD // L, zero_body, 0)

    ones = jnp.ones((L,), jnp.int32)

    def scat_body(i, _):
        idx = ev[pl.ds(pl.multiple_of(i * L, L), L)]
        plsc.addupdate_scatter(hist, [idx], ones)
        return 0

    lax.fori_loop(0, (E // NS) // L, scat_body, 0)
    pltpu.sync_copy(hist, parts_hbm.at[c, s])


def _mm_body(x_ref, w1_ref, wg_ref, parts_ref, h2_ref, binv_ref):
    deg_s = jnp.sum(parts_ref[0].astype(jnp.float32), axis=0)
    a = lax.rsqrt(jnp.maximum(deg_s, 1.0))
    xs = x_ref[...] * a[:, None]
    h1 = jnp.maximum(jnp.dot(xs, w1_ref[...], preferred_element_type=jnp.float32), 0.0)
    hh = jnp.dot(h1, wg_ref[...], preferred_element_type=jnp.float32)
    h2_ref[0] = hh[:, :128]
    h2_ref[1] = hh[:, 128:]
    deg_d = jnp.sum(parts_ref[1].astype(jnp.float32), axis=0)
    binv_ref[...] = lax.rsqrt(jnp.maximum(deg_d, 1.0))[:, None]


def _matmul_stage(x_pad, W1, Wg, parts):
    return pl.pallas_call(
        _mm_body,
        grid=(NPAD // BN2,),
        in_specs=[
            pl.BlockSpec((BN2, D_IN), lambda i: (i, 0)),
            pl.BlockSpec((D_IN, HID), lambda i: (0, 0)),
            pl.BlockSpec((HID, D_OUT), lambda i: (0, 0)),
            pl.BlockSpec((2, NS, BN2), lambda i: (0, 0, i)),
        ],
        out_specs=[
            pl.BlockSpec((2, BN2, 128), lambda i: (0, i, 0)),
            pl.BlockSpec((BN2, 1), lambda i: (i, 0)),
        ],
        out_shape=[
            jax.ShapeDtypeStruct((2, NPAD, 128), jnp.float32),
            jax.ShapeDtypeStruct((NPAD, 1), jnp.float32),
        ],
    )(x_pad, W1, Wg, parts)


@functools.partial(
    pl.kernel,
    out_type=jax.ShapeDtypeStruct((2, NPAD, 128), jnp.float32),
    mesh=_mesh(),
    scratch_types=[
        pltpu.VMEM((NCHUNK, K), jnp.int32),
        pltpu.VMEM((NCHUNK, K), jnp.int32),
        pltpu.VMEM((NCHUNK, K), jnp.int32),
        pltpu.VMEM((K, 128), jnp.float32),
        pltpu.VMEM_SHARED((NPAD, 128), jnp.float32),
        pltpu.SemaphoreType.DMA,
    ],
    compiler_params=pltpu.CompilerParams(needs_layout_passes=False),
)
def _scatter_kernel(h_hbm, src_hbm, dst_hbm, zeros_hbm, acc_hbm,
                    srcv, dstv, gidx, rows, acc, sem):
    c = lax.axis_index("c")
    s = lax.axis_index("s")
    rpt = NPAD // NS

    pltpu.sync_copy(zeros_hbm, acc.at[pl.ds(s * rpt, rpt)])

    pltpu.sync_copy(src_hbm.at[s], srcv)
    pltpu.sync_copy(dst_hbm.at[s], dstv)

    off = c * NPAD

    def add_body(j, _):
        def inner(k, _):
            sl = pl.ds(pl.multiple_of(k * L, L), L)
            gidx[j, sl] = srcv[j, sl] + off
            return 0
        lax.fori_loop(0, K // L, inner, 0)
        return 0

    lax.fori_loop(0, NCHUNK, add_body, 0)

    plsc.subcore_barrier()

    def edge_body(j, _):
        pltpu.async_copy(h_hbm.at[gidx.at[j]], rows, sem).wait()
        pltpu.sync_copy(rows, acc.at[dstv.at[j]], add=True)
        return 0

    lax.fori_loop(0, NCHUNK, edge_body, 0)

    plsc.subcore_barrier()
    pltpu.sync_copy(acc.at[pl.ds(s * rpt, rpt)], acc_hbm.at[c, pl.ds(s * rpt, rpt)])


def _post_body(acc_ref, b_ref, out_ref):
    out_ref[...] = acc_ref[0].astype(jnp.float32) * b_ref[...]


def _post_stage(acc2, binv):
    return pl.pallas_call(
        _post_body,
        grid=(N // 1000, 2),
        in_specs=[
            pl.BlockSpec((1, 1000, 128), lambda i, j: (j, i, 0)),
            pl.BlockSpec((1000, 1), lambda i, j: (i, 0)),
        ],
        out_specs=pl.BlockSpec((1000, 128), lambda i, j: (i, j)),
        out_shape=jax.ShapeDtypeStruct((N, D_OUT), jnp.float32),
    )(acc2, binv)


def kernel(x, adj, W1, Wg):
    src = adj[0]
    dst = adj[1]
    adj16 = adj.reshape(2, NS, E // NS)
    dummy = N + jnp.arange(EPAD - E, dtype=jnp.int32) % (NPAD - N)
    src_r = jnp.concatenate([src, dummy]).reshape(NS, NCHUNK, K)
    dst_r = jnp.concatenate([dst, dummy]).reshape(NS, NCHUNK, K)
    x_pad = jnp.pad(x, ((0, NPAD - N), (0, 0)))
    zeros_blk = jnp.zeros((NPAD // NS, 128), jnp.float32)

    parts = _deg_kernel(adj16)
    h2, binv = _matmul_stage(x_pad, W1, Wg, parts)
    h_flat = h2.reshape(2 * NPAD, 128)
    acc2 = _scatter_kernel(h_flat, src_r, dst_r, zeros_blk)
    out = _post_stage(acc2, binv)
    return out


# speedup vs baseline: 2.4991x; 1.3625x over previous
import functools

import jax
import jax.numpy as jnp
from jax import lax
from jax.experimental import pallas as pl
from jax.experimental.pallas import tpu as pltpu
from jax.experimental.pallas import tpu_sc as plsc

N = 10000
E = 160000
D_IN = 256
HID = 512
D_OUT = 256

NC = 2
NS = 16
L = 16

NPAD = 10240
BN2 = NPAD // 8

K = 128
NCHUNK = 80
NBUF = 2
P = 2
CPP = NCHUNK // P
EPT = NCHUNK * K
EPAD = NS * EPT
JUNK = NPAD - 1

EPH = E // (NC * NS)


def _mesh():
    return plsc.VectorSubcoreMesh(
        core_axis_name="c", subcore_axis_name="s", num_cores=NC, num_subcores=NS
    )


@functools.partial(
    pl.kernel,
    out_type=jax.ShapeDtypeStruct((2, NS, NPAD), jnp.int32),
    mesh=_mesh(),
    scratch_types=[
        pltpu.VMEM((E // NS,), jnp.int32),
        pltpu.VMEM((NPAD,), jnp.int32),
    ],
    compiler_params=pltpu.CompilerParams(needs_layout_passes=False),
)
def _deg_kernel(adj_hbm, parts_hbm, ev, hist):
    c = lax.axis_index("c")
    s = lax.axis_index("s")
    pltpu.sync_copy(adj_hbm.at[c, s], ev)

    zeros = jnp.zeros((L,), jnp.int32)

    def zero_body(i, _):
        hist[pl.ds(pl.multiple_of(i * L, L), L)] = zeros
        return 0

    lax.fori_loop(0, NPAD // L, zero_body, 0)

    ones = jnp.ones((L,), jnp.int32)

    def scat_body(i, _):
        idx = ev[pl.ds(pl.multiple_of(i * L, L), L)]
        plsc.addupdate_scatter(hist, [idx], ones)
        return 0

    lax.fori_loop(0, (E // NS) // L, scat_body, 0)
    pltpu.sync_copy(hist, parts_hbm.at[c, s])


def _mm_body(x_ref, w1_ref, wg_ref, parts_ref, h2_ref, binv_ref):
    deg_s = jnp.sum(parts_ref[0].astype(jnp.float32), axis=0)
    a = lax.rsqrt(jnp.maximum(deg_s, 1.0))
    xs = x_ref[...] * a[:, None]
    h1 = jnp.maximum(jnp.dot(xs, w1_ref[...], preferred_element_type=jnp.float32), 0.0)
    hh = jnp.dot(h1, wg_ref[...], preferred_element_type=jnp.float32)
    h2_ref[0] = hh[:, :128]
    h2_ref[1] = hh[:, 128:]
    deg_d = jnp.sum(parts_ref[1].astype(jnp.float32), axis=0)
    binv_ref[...] = lax.rsqrt(jnp.maximum(deg_d, 1.0))[:, None]


def _matmul_stage(x_pad, W1, Wg, parts):
    return pl.pallas_call(
        _mm_body,
        grid=(NPAD // BN2,),
        in_specs=[
            pl.BlockSpec((BN2, D_IN), lambda i: (i, 0)),
            pl.BlockSpec((D_IN, HID), lambda i: (0, 0)),
            pl.BlockSpec((HID, D_OUT), lambda i: (0, 0)),
            pl.BlockSpec((2, NS, BN2), lambda i: (0, 0, i)),
        ],
        out_specs=[
            pl.BlockSpec((2, BN2, 128), lambda i: (0, i, 0)),
            pl.BlockSpec((BN2, 1), lambda i: (i, 0)),
        ],
        out_shape=[
            jax.ShapeDtypeStruct((2, NPAD, 128), jnp.float32),
            jax.ShapeDtypeStruct((NPAD, 1), jnp.float32),
        ],
    )(x_pad, W1, Wg, parts)


@functools.partial(
    pl.kernel,
    out_type=jax.ShapeDtypeStruct((2, NPAD, 128), jnp.float32),
    mesh=_mesh(),
    scratch_types=[
        pltpu.VMEM((CPP, K), jnp.int32),
        pltpu.VMEM((CPP, K), jnp.int32),
        pltpu.VMEM((NBUF, K, 128), jnp.float32),
        pltpu.VMEM_SHARED((NPAD, 128), jnp.float32),
        pltpu.SemaphoreType.DMA,
        pltpu.SemaphoreType.DMA,
    ],
    compiler_params=pltpu.CompilerParams(needs_layout_passes=False),
)
def _scatter_kernel(h_hbm, src_hbm, dst_hbm, zeros_hbm, acc_hbm,
                    srcv, dstv, rows, acc, sem0, sem1):
    c = lax.axis_index("c")
    s = lax.axis_index("s")
    rpt = NPAD // NS
    sems = (sem0, sem1)

    pltpu.sync_copy(zeros_hbm, acc.at[pl.ds(s * rpt, rpt)])
    plsc.subcore_barrier()

    off = c * NPAD

    for p in range(P):
        pltpu.sync_copy(src_hbm.at[s, p], srcv)
        pltpu.sync_copy(dst_hbm.at[s, p], dstv)

        def add_body(j, _):
            def inner(k, _):
                sl = pl.ds(pl.multiple_of(k * L, L), L)
                srcv[j, sl] = srcv[j, sl] + off
                return 0
            lax.fori_loop(0, K // L, inner, 0)
            return 0

        lax.fori_loop(0, CPP, add_body, 0)

        for b in range(NBUF):
            pltpu.async_copy(h_hbm.at[srcv.at[b]], rows.at[b], sems[b])

        def edge_body(i, _):
            for b in range(NBUF):
                j = i * NBUF + b
                pltpu.make_async_copy(
                    h_hbm.at[srcv.at[j]], rows.at[b], sems[b]).wait()
                pltpu.sync_copy(rows.at[b], acc.at[dstv.at[j]], add=True)

                @pl.when(j + NBUF < CPP)
                def _():
                    pltpu.async_copy(h_hbm.at[srcv.at[j + NBUF]], rows.at[b], sems[b])
            return 0

        lax.fori_loop(0, CPP // NBUF, edge_body, 0)

    plsc.subcore_barrier()
    pltpu.sync_copy(acc.at[pl.ds(s * rpt, rpt)], acc_hbm.at[c, pl.ds(s * rpt, rpt)])


def _post_body(acc_ref, b_ref, out_ref):
    out_ref[...] = acc_ref[0].astype(jnp.float32) * b_ref[...]


def _post_stage(acc2, binv):
    return pl.pallas_call(
        _post_body,
        grid=(N // 1000, 2),
        in_specs=[
            pl.BlockSpec((1, 1000, 128), lambda i, j: (j, i, 0)),
            pl.BlockSpec((1000, 1), lambda i, j: (i, 0)),
        ],
        out_specs=pl.BlockSpec((1000, 128), lambda i, j: (i, j)),
        out_shape=jax.ShapeDtypeStruct((N, D_OUT), jnp.float32),
    )(acc2, binv)


def kernel(x, adj, W1, Wg):
    src = adj[0]
    dst = adj[1]
    adj16 = adj.reshape(2, NS, E // NS)
    dummy = N + jnp.arange(EPAD - E, dtype=jnp.int32) % (NPAD - N)
    src_r = jnp.concatenate([src, dummy]).reshape(NS, P, CPP, K)
    dst_r = jnp.concatenate([dst, dummy]).reshape(NS, P, CPP, K)
    x_pad = jnp.pad(x, ((0, NPAD - N), (0, 0)))
    zeros_blk = jnp.zeros((NPAD // NS, 128), jnp.float32)

    parts = _deg_kernel(adj16)
    h2, binv = _matmul_stage(x_pad, W1, Wg, parts)
    h_flat = h2.reshape(2 * NPAD, 128)
    acc2 = _scatter_kernel(h_flat, src_r, dst_r, zeros_blk)
    out = _post_stage(acc2, binv)
    return out


# speedup vs baseline: 2.7094x; 1.0842x over previous
import functools

import jax
import jax.numpy as jnp
from jax import lax
from jax.experimental import pallas as pl
from jax.experimental.pallas import tpu as pltpu
from jax.experimental.pallas import tpu_sc as plsc

N = 10000
E = 160000
D_IN = 256
HID = 512
D_OUT = 256

NC = 2
NS = 16
L = 16

NPAD = 10240
BN2 = NPAD // 8

K = 128
NCHUNK = 80
NBUF = 2
P = 2
CPP = NCHUNK // P
EPT = NCHUNK * K
EPAD = NS * EPT
JUNK = NPAD - 1

EPH = E // (NC * NS)


def _mesh():
    return plsc.VectorSubcoreMesh(
        core_axis_name="c", subcore_axis_name="s", num_cores=NC, num_subcores=NS
    )


@functools.partial(
    pl.kernel,
    out_type=jax.ShapeDtypeStruct((2, NS, NPAD), jnp.int32),
    mesh=_mesh(),
    scratch_types=[
        pltpu.VMEM((E // NS,), jnp.int32),
        pltpu.VMEM((NPAD,), jnp.int32),
    ],
    compiler_params=pltpu.CompilerParams(needs_layout_passes=False),
)
def _deg_kernel(adj_hbm, parts_hbm, ev, hist):
    c = lax.axis_index("c")
    s = lax.axis_index("s")
    pltpu.sync_copy(adj_hbm.at[c, s], ev)

    zeros = jnp.zeros((L,), jnp.int32)

    def zero_body(i, _):
        hist[pl.ds(pl.multiple_of(i * L, L), L)] = zeros
        return 0

    lax.fori_loop(0, NPAD // L, zero_body, 0)

    ones = jnp.ones((L,), jnp.int32)

    def scat_body(i, _):
        idx = ev[pl.ds(pl.multiple_of(i * L, L), L)]
        plsc.addupdate_scatter(hist, [idx], ones)
        return 0

    lax.fori_loop(0, (E // NS) // L, scat_body, 0)
    pltpu.sync_copy(hist, parts_hbm.at[c, s])


def _mm_body(x_ref, w1_ref, wg_ref, parts_ref, h2_ref, binv_ref):
    deg_s = jnp.sum(parts_ref[0].astype(jnp.float32), axis=0)
    a = lax.rsqrt(jnp.maximum(deg_s, 1.0))
    xs = x_ref[...] * a[:, None]
    h1 = jnp.maximum(jnp.dot(xs, w1_ref[...], preferred_element_type=jnp.float32), 0.0)
    hh = jnp.dot(h1, wg_ref[...], preferred_element_type=jnp.float32)
    h2_ref[0] = hh[:, :128]
    h2_ref[1] = hh[:, 128:]
    deg_d = jnp.sum(parts_ref[1].astype(jnp.float32), axis=0)
    binv_ref[...] = lax.rsqrt(jnp.maximum(deg_d, 1.0))[:, None]


def _matmul_stage(x_pad, W1, Wg, parts):
    return pl.pallas_call(
        _mm_body,
        grid=(NPAD // BN2,),
        in_specs=[
            pl.BlockSpec((BN2, D_IN), lambda i: (i, 0)),
            pl.BlockSpec((D_IN, HID), lambda i: (0, 0)),
            pl.BlockSpec((HID, D_OUT), lambda i: (0, 0)),
            pl.BlockSpec((2, NS, BN2), lambda i: (0, 0, i)),
        ],
        out_specs=[
            pl.BlockSpec((2, BN2, 128), lambda i: (0, i, 0)),
            pl.BlockSpec((BN2, 1), lambda i: (i, 0)),
        ],
        out_shape=[
            jax.ShapeDtypeStruct((2, NPAD, 128), jnp.float32),
            jax.ShapeDtypeStruct((NPAD, 1), jnp.float32),
        ],
    )(x_pad, W1, Wg, parts)


@functools.partial(
    pl.kernel,
    out_type=jax.ShapeDtypeStruct((2, NPAD, 128), jnp.float32),
    mesh=_mesh(),
    scratch_types=[
        pltpu.VMEM((CPP, K), jnp.int32),
        pltpu.VMEM((CPP, K), jnp.int32),
        pltpu.VMEM((NBUF, K, 128), jnp.float32),
        pltpu.VMEM_SHARED((NPAD, 128), jnp.float32),
        pltpu.SemaphoreType.DMA,
        pltpu.SemaphoreType.DMA,
    ],
    compiler_params=pltpu.CompilerParams(needs_layout_passes=False),
)
def _scatter_kernel(h_hbm, src_hbm, dst_hbm, zeros_hbm, acc_hbm,
                    srcv, dstv, rows, acc, sem0, sem1):
    c = lax.axis_index("c")
    s = lax.axis_index("s")
    rpt = NPAD // NS
    sems = (sem0, sem1)

    pltpu.sync_copy(zeros_hbm, acc.at[pl.ds(s * rpt, rpt)])
    plsc.subcore_barrier()

    off = c * NPAD

    for p in range(P):
        pltpu.sync_copy(src_hbm.at[s, p], srcv)
        pltpu.sync_copy(dst_hbm.at[s, p], dstv)

        def add_body(j, _):
            def inner(k, _):
                sl = pl.ds(pl.multiple_of(k * L, L), L)
                srcv[j, sl] = srcv[j, sl] + off
                return 0
            lax.fori_loop(0, K // L, inner, 0)
            return 0

        lax.fori_loop(0, CPP, add_body, 0)

        for b in range(NBUF):
            pltpu.async_copy(h_hbm.at[srcv.at[b]], rows.at[b], sems[b])

        def edge_body(i, _):
            for b in range(NBUF):
                j = i * NBUF + b
                pltpu.make_async_copy(
                    h_hbm.at[srcv.at[j]], rows.at[b], sems[b]).wait()
                pltpu.sync_copy(rows.at[b], acc.at[dstv.at[j]], add=True)

                @pl.when(j + NBUF < CPP)
                def _():
                    pltpu.async_copy(h_hbm.at[srcv.at[j + NBUF]], rows.at[b], sems[b])
            return 0

        lax.fori_loop(0, CPP // NBUF, edge_body, 0)

    plsc.subcore_barrier()
    pltpu.sync_copy(acc.at[pl.ds(s * rpt, rpt)], acc_hbm.at[c, pl.ds(s * rpt, rpt)])


def _post_body(acc_ref, b_ref, out_ref):
    out_ref[:, :128] = acc_ref[0] * b_ref[...]
    out_ref[:, 128:] = acc_ref[1] * b_ref[...]


def _post_stage(acc2, binv):
    return pl.pallas_call(
        _post_body,
        grid=(N // 2000,),
        in_specs=[
            pl.BlockSpec((2, 2000, 128), lambda i: (0, i, 0)),
            pl.BlockSpec((2000, 1), lambda i: (i, 0)),
        ],
        out_specs=pl.BlockSpec((2000, D_OUT), lambda i: (i, 0)),
        out_shape=jax.ShapeDtypeStruct((N, D_OUT), jnp.float32),
    )(acc2, binv)


def kernel(x, adj, W1, Wg):
    src = adj[0]
    dst = adj[1]
    adj16 = adj.reshape(2, NS, E // NS)
    dummy = N + jnp.arange(EPAD - E, dtype=jnp.int32) % (NPAD - N)
    src_r = jnp.concatenate([src, dummy]).reshape(NS, P, CPP, K)
    dst_r = jnp.concatenate([dst, dummy]).reshape(NS, P, CPP, K)
    zeros_blk = jnp.zeros((NPAD // NS, 128), jnp.float32)

    parts = _deg_kernel(adj16)
    h2, binv = _matmul_stage(x, W1, Wg, parts)
    h_flat = h2.reshape(2 * NPAD, 128)
    acc2 = _scatter_kernel(h_flat, src_r, dst_r, zeros_blk)
    out = _post_stage(acc2, binv)
    return out


# speedup vs baseline: 2.7361x; 1.0098x over previous
import functools

import jax
import jax.numpy as jnp
from jax import lax
from jax.experimental import pallas as pl
from jax.experimental.pallas import tpu as pltpu
from jax.experimental.pallas import tpu_sc as plsc

N = 10000
E = 160000
D_IN = 256
HID = 512
D_OUT = 256

NC = 2
NS = 16
L = 16

NPAD = 10240
BN2 = NPAD // 8

K = 128
NCHUNK = 80
NBUF = 2
P = 2
CPP = NCHUNK // P
EPT = NCHUNK * K
EPAD = NS * EPT
JUNK = NPAD - 1

EPH = E // (NC * NS)


def _mesh():
    return plsc.VectorSubcoreMesh(
        core_axis_name="c", subcore_axis_name="s", num_cores=NC, num_subcores=NS
    )


@functools.partial(
    pl.kernel,
    out_type=jax.ShapeDtypeStruct((2, NS, NPAD), jnp.int32),
    mesh=_mesh(),
    scratch_types=[
        pltpu.VMEM((E // NS,), jnp.int32),
        pltpu.VMEM((NPAD,), jnp.int32),
    ],
    compiler_params=pltpu.CompilerParams(needs_layout_passes=False),
)
def _deg_kernel(adj_hbm, parts_hbm, ev, hist):
    c = lax.axis_index("c")
    s = lax.axis_index("s")
    pltpu.sync_copy(adj_hbm.at[c, s], ev)

    zeros = jnp.zeros((L,), jnp.int32)

    def zero_body(i, _):
        hist[pl.ds(pl.multiple_of(i * L, L), L)] = zeros
        return 0

    lax.fori_loop(0, NPAD // L, zero_body, 0)

    ones = jnp.ones((L,), jnp.int32)

    def scat_body(i, _):
        idx = ev[pl.ds(pl.multiple_of(i * L, L), L)]
        plsc.addupdate_scatter(hist, [idx], ones)
        return 0

    lax.fori_loop(0, (E // NS) // L, scat_body, 0)
    pltpu.sync_copy(hist, parts_hbm.at[c, s])


def _mm_body(x_ref, w1_ref, wg_ref, parts_ref, h2_ref, binv_ref):
    deg_s = jnp.sum(parts_ref[0].astype(jnp.float32), axis=0)
    a = lax.rsqrt(jnp.maximum(deg_s, 1.0))
    xs = x_ref[...] * a[:, None]
    h1 = jnp.maximum(jnp.dot(xs, w1_ref[...], preferred_element_type=jnp.float32), 0.0)
    hh = jnp.dot(h1, wg_ref[...], preferred_element_type=jnp.float32)
    h2_ref[0] = hh[:, :128]
    h2_ref[1] = hh[:, 128:]
    deg_d = jnp.sum(parts_ref[1].astype(jnp.float32), axis=0)
    binv_ref[...] = lax.rsqrt(jnp.maximum(deg_d, 1.0))[:, None]


def _matmul_stage(x_pad, W1, Wg, parts):
    return pl.pallas_call(
        _mm_body,
        grid=(NPAD // BN2,),
        in_specs=[
            pl.BlockSpec((BN2, D_IN), lambda i: (i, 0)),
            pl.BlockSpec((D_IN, HID), lambda i: (0, 0)),
            pl.BlockSpec((HID, D_OUT), lambda i: (0, 0)),
            pl.BlockSpec((2, NS, BN2), lambda i: (0, 0, i)),
        ],
        out_specs=[
            pl.BlockSpec((2, BN2, 128), lambda i: (0, i, 0)),
            pl.BlockSpec((BN2, 1), lambda i: (i, 0)),
        ],
        out_shape=[
            jax.ShapeDtypeStruct((2, NPAD, 128), jnp.float32),
            jax.ShapeDtypeStruct((NPAD, 1), jnp.float32),
        ],
    )(x_pad, W1, Wg, parts)


@functools.partial(
    pl.kernel,
    out_type=jax.ShapeDtypeStruct((2, NPAD, 128), jnp.float32),
    mesh=_mesh(),
    scratch_types=[
        pltpu.VMEM((CPP, K), jnp.int32),
        pltpu.VMEM((CPP, K), jnp.int32),
        pltpu.VMEM((NBUF, K, 128), jnp.float32),
        pltpu.VMEM_SHARED((NPAD, 128), jnp.float32),
        pltpu.SemaphoreType.DMA,
        pltpu.SemaphoreType.DMA,
        pltpu.SemaphoreType.DMA,
    ],
    compiler_params=pltpu.CompilerParams(needs_layout_passes=False),
)
def _scatter_kernel(h_hbm, src_hbm, dst_hbm, zeros_hbm, acc_hbm,
                    srcv, dstv, rows, acc, sem0, sem1, zsem):
    c = lax.axis_index("c")
    s = lax.axis_index("s")
    rpt = NPAD // NS
    sems = (sem0, sem1)

    zcp = pltpu.async_copy(zeros_hbm, acc.at[pl.ds(s * rpt, rpt)], zsem)

    off = c * NPAD

    for p in range(P):
        pltpu.sync_copy(src_hbm.at[s, p], srcv)
        pltpu.sync_copy(dst_hbm.at[s, p], dstv)

        def add_body(j, _):
            def inner(k, _):
                sl = pl.ds(pl.multiple_of(k * L, L), L)
                srcv[j, sl] = srcv[j, sl] + off
                return 0
            lax.fori_loop(0, K // L, inner, 0)
            return 0

        lax.fori_loop(0, CPP, add_body, 0)

        if p == 0:
            zcp.wait()
            plsc.subcore_barrier()

        for b in range(NBUF):
            pltpu.async_copy(h_hbm.at[srcv.at[b]], rows.at[b], sems[b])

        def edge_body(i, _):
            for b in range(NBUF):
                j = i * NBUF + b
                pltpu.make_async_copy(
                    h_hbm.at[srcv.at[j]], rows.at[b], sems[b]).wait()
                pltpu.sync_copy(rows.at[b], acc.at[dstv.at[j]], add=True)

                @pl.when(j + NBUF < CPP)
                def _():
                    pltpu.async_copy(h_hbm.at[srcv.at[j + NBUF]], rows.at[b], sems[b])
            return 0

        lax.fori_loop(0, CPP // NBUF, edge_body, 0)

    plsc.subcore_barrier()
    pltpu.sync_copy(acc.at[pl.ds(s * rpt, rpt)], acc_hbm.at[c, pl.ds(s * rpt, rpt)])


def _post_body(acc_ref, b_ref, out_ref):
    out_ref[:, :128] = acc_ref[0] * b_ref[...]
    out_ref[:, 128:] = acc_ref[1] * b_ref[...]


def _post_stage(acc2, binv):
    return pl.pallas_call(
        _post_body,
        grid=(N // 2000,),
        in_specs=[
            pl.BlockSpec((2, 2000, 128), lambda i: (0, i, 0)),
            pl.BlockSpec((2000, 1), lambda i: (i, 0)),
        ],
        out_specs=pl.BlockSpec((2000, D_OUT), lambda i: (i, 0)),
        out_shape=jax.ShapeDtypeStruct((N, D_OUT), jnp.float32),
    )(acc2, binv)


def kernel(x, adj, W1, Wg):
    src = adj[0]
    dst = adj[1]
    adj16 = adj.reshape(2, NS, E // NS)
    dummy = N + jnp.arange(EPAD - E, dtype=jnp.int32) % (NPAD - N)
    src_r = jnp.concatenate([src, dummy]).reshape(NS, P, CPP, K)
    dst_r = jnp.concatenate([dst, dummy]).reshape(NS, P, CPP, K)
    zeros_blk = jnp.zeros((NPAD // NS, 128), jnp.float32)

    parts = _deg_kernel(adj16)
    h2, binv = _matmul_stage(x, W1, Wg, parts)
    h_flat = h2.reshape(2 * NPAD, 128)
    acc2 = _scatter_kernel(h_flat, src_r, dst_r, zeros_blk)
    out = _post_stage(acc2, binv)
    return out


# speedup vs baseline: 2.7418x; 1.0021x over previous
import functools

import jax
import jax.numpy as jnp
from jax import lax
from jax.experimental import pallas as pl
from jax.experimental.pallas import tpu as pltpu
from jax.experimental.pallas import tpu_sc as plsc

N = 10000
E = 160000
D_IN = 256
HID = 512
D_OUT = 256

NC = 2
NS = 16
L = 16

NPAD = 10240
BN2 = NPAD // 8

K = 128
NCHUNK = 80
NBUF = 2
P = 2
CPP = NCHUNK // P
EPT = NCHUNK * K
EPAD = NS * EPT
JUNK = NPAD - 1


def _mesh():
    return plsc.VectorSubcoreMesh(
        core_axis_name="c", subcore_axis_name="s", num_cores=NC, num_subcores=NS
    )


@functools.partial(
    pl.kernel,
    out_type=jax.ShapeDtypeStruct((2, NS, NPAD), jnp.int32),
    mesh=_mesh(),
    scratch_types=[
        pltpu.VMEM((E // NS,), jnp.int32),
        pltpu.VMEM((NPAD,), jnp.int32),
    ],
    compiler_params=pltpu.CompilerParams(needs_layout_passes=False),
)
def _deg_kernel(adj_hbm, parts_hbm, ev, hist):
    c = lax.axis_index("c")
    s = lax.axis_index("s")
    pltpu.sync_copy(adj_hbm.at[c, s], ev)

    zeros = jnp.zeros((L,), jnp.int32)

    def zero_body(i, _):
        hist[pl.ds(pl.multiple_of(i * L, L), L)] = zeros
        return 0

    lax.fori_loop(0, NPAD // L, zero_body, 0)

    ones = jnp.ones((L,), jnp.int32)

    def scat_body(i, _):
        idx = ev[pl.ds(pl.multiple_of(i * L, L), L)]
        plsc.addupdate_scatter(hist, [idx], ones)
        return 0

    lax.fori_loop(0, (E // NS) // L, scat_body, 0)
    pltpu.sync_copy(hist, parts_hbm.at[c, s])


def _mm_body(x_ref, w1_ref, wg_ref, parts_ref, h2_ref, binv_ref):
    deg_s = jnp.sum(parts_ref[0].astype(jnp.float32), axis=0)
    a = lax.rsqrt(jnp.maximum(deg_s, 1.0))
    xs = x_ref[...] * a[:, None]
    h1 = jnp.maximum(jnp.dot(xs, w1_ref[...], preferred_element_type=jnp.float32), 0.0)
    hh = jnp.dot(h1, wg_ref[...], preferred_element_type=jnp.float32)
    h2_ref[0] = hh[:, :128]
    h2_ref[1] = hh[:, 128:]
    deg_d = jnp.sum(parts_ref[1].astype(jnp.float32), axis=0)
    binv_ref[...] = lax.rsqrt(jnp.maximum(deg_d, 1.0))[:, None]


def _matmul_stage(x_pad, W1, Wg, parts):
    return pl.pallas_call(
        _mm_body,
        grid=(NPAD // BN2,),
        in_specs=[
            pl.BlockSpec((BN2, D_IN), lambda i: (i, 0)),
            pl.BlockSpec((D_IN, HID), lambda i: (0, 0)),
            pl.BlockSpec((HID, D_OUT), lambda i: (0, 0)),
            pl.BlockSpec((2, NS, BN2), lambda i: (0, 0, i)),
        ],
        out_specs=[
            pl.BlockSpec((2, BN2, 128), lambda i: (0, i, 0)),
            pl.BlockSpec((BN2, 1), lambda i: (i, 0)),
        ],
        out_shape=[
            jax.ShapeDtypeStruct((2, NPAD, 128), jnp.float32),
            jax.ShapeDtypeStruct((NPAD, 1), jnp.float32),
        ],
    )(x_pad, W1, Wg, parts)


@functools.partial(
    pl.kernel,
    out_type=jax.ShapeDtypeStruct((2, NPAD, 128), jnp.float32),
    mesh=_mesh(),
    scratch_types=[
        pltpu.VMEM((CPP, K), jnp.int32),
        pltpu.VMEM((CPP, K), jnp.int32),
        pltpu.VMEM((NBUF, K, 128), jnp.float32),
        pltpu.VMEM_SHARED((NPAD, 128), jnp.float32),
        pltpu.SemaphoreType.DMA,
        pltpu.SemaphoreType.DMA,
        pltpu.SemaphoreType.DMA,
    ],
    compiler_params=pltpu.CompilerParams(needs_layout_passes=False),
)
def _scatter_kernel(h_hbm, src_hbm, dst_hbm, zeros_hbm, acc_hbm,
                    srcv, dstv, rows, acc, sem0, sem1, zsem):
    c = lax.axis_index("c")
    s = lax.axis_index("s")
    rpt = NPAD // NS
    sems = (sem0, sem1)

    zcp = pltpu.async_copy(zeros_hbm, acc.at[pl.ds(s * rpt, rpt)], zsem)

    off = c * NPAD

    for p in range(P):
        pltpu.sync_copy(src_hbm.at[s, p], srcv)
        pltpu.sync_copy(dst_hbm.at[s, p], dstv)

        def add_body(j, _):
            def inner(k, _):
                sl = pl.ds(pl.multiple_of(k * L, L), L)
                srcv[j, sl] = srcv[j, sl] + off
                return 0
            lax.fori_loop(0, K // L, inner, 0)
            return 0

        lax.fori_loop(0, CPP, add_body, 0)

        if p == 0:
            zcp.wait()
            plsc.subcore_barrier()

        for b in range(NBUF):
            pltpu.async_copy(h_hbm.at[srcv.at[b]], rows.at[b], sems[b])

        def edge_body(i, _):
            for b in range(NBUF):
                j = i * NBUF + b
                pltpu.make_async_copy(
                    h_hbm.at[srcv.at[j]], rows.at[b], sems[b]).wait()
                pltpu.sync_copy(rows.at[b], acc.at[dstv.at[j]], add=True)

                @pl.when(j + NBUF < CPP)
                def _():
                    pltpu.async_copy(h_hbm.at[srcv.at[j + NBUF]], rows.at[b], sems[b])
            return 0

        lax.fori_loop(0, CPP // NBUF, edge_body, 0)

    plsc.subcore_barrier()
    pltpu.sync_copy(acc.at[pl.ds(s * rpt, rpt)], acc_hbm.at[c, pl.ds(s * rpt, rpt)])


def _post_body(acc_ref, b_ref, out_ref):
    out_ref[:, :128] = acc_ref[0] * b_ref[...]
    out_ref[:, 128:] = acc_ref[1] * b_ref[...]


def _post_stage(acc2, binv):
    return pl.pallas_call(
        _post_body,
        grid=(N // 2000,),
        in_specs=[
            pl.BlockSpec((2, 2000, 128), lambda i: (0, i, 0)),
            pl.BlockSpec((2000, 1), lambda i: (i, 0)),
        ],
        out_specs=pl.BlockSpec((2000, D_OUT), lambda i: (i, 0)),
        out_shape=jax.ShapeDtypeStruct((N, D_OUT), jnp.float32),
    )(acc2, binv)


def kernel(x, adj, W1, Wg):
    src = adj[0]
    dst = adj[1]
    adj16 = adj.reshape(2, NS, E // NS)
    dummy = N + jnp.arange(EPAD - E, dtype=jnp.int32) % (NPAD - N)
    src_r = jnp.concatenate([src, dummy]).reshape(NS, P, CPP, K)
    dst_r = jnp.concatenate([dst, dummy]).reshape(NS, P, CPP, K)
    zeros_blk = jnp.zeros((NPAD // NS, 128), jnp.float32)

    parts = _deg_kernel(adj16)
    h2, binv = _matmul_stage(x, W1, Wg, parts)
    h_flat = h2.reshape(2 * NPAD, 128)
    acc2 = _scatter_kernel(h_flat, src_r, dst_r, zeros_blk)
    out = _post_stage(acc2, binv)
    return out
